# Initial kernel scaffold; baseline (speedup 1.0000x reference)
import functools

import jax
import jax.numpy as jnp
from jax import lax
from jax.experimental import pallas as pl
from jax.experimental.pallas import tpu as pltpu
from jax.experimental.pallas import tpu_sc as plsc

V = 10000
E = 320000
H = 128
NPROP = 5
VP = 10240
NW = 32
EPW = E // NW
K = 400
NCH = EPW // K
RPT = VP // 16
BM = 1024
BE = 2000

f32 = jnp.float32


def _sc_mesh():
    return plsc.VectorSubcoreMesh(core_axis_name="c", subcore_axis_name="s")


def _gather_body(ap, bp, ii, io, m1, ii_v, io_v, a_v, b_v, o_v, s1, s2):
    wid = lax.axis_index("s") * 2 + lax.axis_index("c")
    for ci in range(NCH):
        base = wid * EPW + ci * K
        pltpu.sync_copy(ii.at[pl.ds(base, K)], ii_v)
        pltpu.sync_copy(io.at[pl.ds(base, K)], io_v)
        ca = pltpu.async_copy(ap.at[ii_v], a_v, s1)
        cb = pltpu.async_copy(bp.at[io_v], b_v, s2)
        ca.wait()
        cb.wait()

        def row(r, carry):
            for cc in range(4):
                sl = pl.ds(16 * cc, 16)
                o_v[r, sl] = a_v[r, sl] + b_v[r, sl]
            return carry

        lax.fori_loop(0, K, row, 0)
        pltpu.sync_copy(o_v, m1.at[pl.ds(base, K)])


def _sc_gather(ap, bp, ii, io):
    gk = pl.kernel(
        _gather_body,
        out_type=jax.ShapeDtypeStruct((E, 64), f32),
        mesh=_sc_mesh(),
        scratch_types=[
            pltpu.VMEM((K,), jnp.int32),
            pltpu.VMEM((K,), jnp.int32),
            pltpu.VMEM((K, 64), f32),
            pltpu.VMEM((K, 64), f32),
            pltpu.VMEM((K, 64), f32),
            pltpu.SemaphoreType.DMA,
            pltpu.SemaphoreType.DMA,
        ],
    )
    return gk(ap, bp, ii, io)


def _zero_fill(buf, rows):
    def zrow(r, carry):
        for cc in range(buf.shape[1] // 16):
            buf[r, pl.ds(16 * cc, 16)] = jnp.zeros((16,), f32)
        return carry

    lax.fori_loop(0, rows, zrow, 0)


def _scatter_body_deg(m2, io, s2o, dego, m2_v, io_v, z_v, ones_v, zd_v, S_sh, D_sh):
    sid = lax.axis_index("s")
    cid = lax.axis_index("c")
    wid = sid * 2 + cid
    _zero_fill(z_v, 64)
    _zero_fill(zd_v, 64)

    def orow(r, carry):
        ones_v[r, pl.ds(0, 16)] = jnp.ones((16,), f32)
        return carry

    lax.fori_loop(0, K, orow, 0)
    for i in range(RPT // 64):
        pltpu.sync_copy(z_v, S_sh.at[pl.ds(sid * RPT + i * 64, 64)])
        pltpu.sync_copy(zd_v, D_sh.at[pl.ds(sid * RPT + i * 64, 64)])
    plsc.subcore_barrier()
    for ci in range(NCH):
        base = wid * EPW + ci * K
        pltpu.sync_copy(m2.at[pl.ds(base, K)], m2_v)
        pltpu.sync_copy(io.at[pl.ds(base, K)], io_v)
        pltpu.sync_copy(m2_v, S_sh.at[io_v], add=True)
        pltpu.sync_copy(ones_v, D_sh.at[io_v], add=True)
    plsc.subcore_barrier()
    pltpu.sync_copy(S_sh.at[pl.ds(sid * RPT, RPT)], s2o.at[cid, pl.ds(sid * RPT, RPT)])
    pltpu.sync_copy(D_sh.at[pl.ds(sid * RPT, RPT)], dego.at[cid, pl.ds(sid * RPT, RPT)])


def _scatter_body(m2, io, s2o, m2_v, io_v, z_v, S_sh):
    sid = lax.axis_index("s")
    cid = lax.axis_index("c")
    wid = sid * 2 + cid
    _zero_fill(z_v, 64)
    for i in range(RPT // 64):
        pltpu.sync_copy(z_v, S_sh.at[pl.ds(sid * RPT + i * 64, 64)])
    plsc.subcore_barrier()
    for ci in range(NCH):
        base = wid * EPW + ci * K
        pltpu.sync_copy(m2.at[pl.ds(base, K)], m2_v)
        pltpu.sync_copy(io.at[pl.ds(base, K)], io_v)
        pltpu.sync_copy(m2_v, S_sh.at[io_v], add=True)
    plsc.subcore_barrier()
    pltpu.sync_copy(S_sh.at[pl.ds(sid * RPT, RPT)], s2o.at[cid, pl.ds(sid * RPT, RPT)])


def _sc_scatter_deg(m2, io):
    sk = pl.kernel(
        _scatter_body_deg,
        out_type=(
            jax.ShapeDtypeStruct((2, VP, 64), f32),
            jax.ShapeDtypeStruct((2, VP, 16), f32),
        ),
        mesh=_sc_mesh(),
        scratch_types=[
            pltpu.VMEM((K, 64), f32),
            pltpu.VMEM((K,), jnp.int32),
            pltpu.VMEM((64, 64), f32),
            pltpu.VMEM((K, 16), f32),
            pltpu.VMEM((64, 16), f32),
            pltpu.VMEM_SHARED((VP, 64), f32),
            pltpu.VMEM_SHARED((VP, 16), f32),
        ],
    )
    return sk(m2, io)


def _sc_scatter(m2, io):
    sk = pl.kernel(
        _scatter_body,
        out_type=jax.ShapeDtypeStruct((2, VP, 64), f32),
        mesh=_sc_mesh(),
        scratch_types=[
            pltpu.VMEM((K, 64), f32),
            pltpu.VMEM((K,), jnp.int32),
            pltpu.VMEM((64, 64), f32),
            pltpu.VMEM_SHARED((VP, 64), f32),
        ],
    )
    return sk(m2, io)


def _edge_body(m1_ref, j_ref, w2_ref, b2_ref, w_ref, o_ref):
    x = m1_ref[...] + j_ref[...] * w_ref[...]
    x = jnp.maximum(x, 0.0)
    y = lax.dot_general(x, w2_ref[...], (((1,), (0,)), ((), ())),
                        preferred_element_type=f32) + b2_ref[...]
    o_ref[...] = jnp.maximum(y, 0.0)


def _tc_edge(m1raw, J, W2T, b2r, wr):
    return pl.pallas_call(
        _edge_body,
        grid=(E // BE,),
        in_specs=[
            pl.BlockSpec((BE, 64), lambda i: (i, 0)),
            pl.BlockSpec((BE, 1), lambda i: (i, 0)),
            pl.BlockSpec((64, 64), lambda i: (0, 0)),
            pl.BlockSpec((1, 64), lambda i: (0, 0)),
            pl.BlockSpec((1, 64), lambda i: (0, 0)),
        ],
        out_specs=pl.BlockSpec((BE, 64), lambda i: (i, 0)),
        out_shape=jax.ShapeDtypeStruct((E, 64), f32),
    )(m1raw, J, W2T, b2r, wr)


def _node_body(s2_ref, st_ref, dvec_ref, ab_ref, bb_ref, ob_ref, t_ref,
               w3t_ref, wiht_ref, whht_ref, bih_ref, bhh_ref,
               o1st_ref, o2t_ref, ob2_ref, o3tp_ref, w1at_ref, w1bt_ref,
               stn_ref, ap_ref, bp_ref, y_ref, l_ref):
    i = pl.program_id(0)
    s = s2_ref[0] + s2_ref[1]
    msg = lax.dot_general(s, w3t_ref[...], (((1,), (0,)), ((), ())),
                          preferred_element_type=f32) + dvec_ref[...]
    st = st_ref[...]
    gi = lax.dot_general(msg, wiht_ref[...], (((1,), (0,)), ((), ())),
                         preferred_element_type=f32) + bih_ref[...]
    gh = lax.dot_general(st, whht_ref[...], (((1,), (0,)), ((), ())),
                         preferred_element_type=f32) + bhh_ref[...]
    r = jax.nn.sigmoid(gi[:, 0:128] + gh[:, 0:128])
    z = jax.nn.sigmoid(gi[:, 128:256] + gh[:, 128:256])
    n = jnp.tanh(gi[:, 256:384] + r * gh[:, 256:384])
    stn = (1.0 - z) * n + z * st
    stn_ref[...] = stn
    o1 = lax.dot_general(stn, o1st_ref[...], (((1,), (0,)), ((), ())),
                         preferred_element_type=f32) + ob_ref[...]
    o1 = jnp.maximum(o1, 0.0)
    o2 = lax.dot_general(o1, o2t_ref[...], (((1,), (0,)), ((), ())),
                         preferred_element_type=f32) + ob2_ref[...]
    o2 = jnp.maximum(o2, 0.0)
    l01 = lax.dot_general(o2, o3tp_ref[...], (((1,), (0,)), ((), ())),
                          preferred_element_type=f32)
    l0 = l01[:, 0:1]
    l1 = l01[:, 1:2]
    m = jnp.maximum(l0, l1)
    lse = m + jnp.log(jnp.exp(l0 - m) + jnp.exp(l1 - m))
    y_ref[...] = jnp.exp(l0 - lse)
    ll = jnp.concatenate([l0 - lse, l1 - lse], axis=1)
    d = ll - jnp.log(t_ref[...])
    rows = i * BM + lax.broadcasted_iota(jnp.int32, (BM, 2), 0)
    sq = jnp.where(rows < V, d * d, 0.0)
    part = jnp.sum(sq)

    @pl.when(i == 0)
    def _():
        l_ref[0, 0] = 0.0

    l_ref[0, 0] += part
    ap_ref[...] = lax.dot_general(stn, w1at_ref[...], (((1,), (0,)), ((), ())),
                                  preferred_element_type=f32) + ab_ref[...]
    bp_ref[...] = lax.dot_general(stn, w1bt_ref[...], (((1,), (0,)), ((), ())),
                                  preferred_element_type=f32) + bb_ref[...]


def _tc_node(s2, st, dvec, abias, bbias, obias, tpad, W3T, WihT, WhhT,
             bihr, bhhr, O1sT, O2T, ob2r, O3Tp, W1aT, W1bT):
    return pl.pallas_call(
        _node_body,
        grid=(VP // BM,),
        in_specs=[
            pl.BlockSpec((2, BM, 64), lambda i: (0, i, 0)),
            pl.BlockSpec((BM, 128), lambda i: (i, 0)),
            pl.BlockSpec((BM, 128), lambda i: (i, 0)),
            pl.BlockSpec((BM, 64), lambda i: (i, 0)),
            pl.BlockSpec((BM, 64), lambda i: (i, 0)),
            pl.BlockSpec((BM, 64), lambda i: (i, 0)),
            pl.BlockSpec((BM, 2), lambda i: (i, 0)),
            pl.BlockSpec((64, 128), lambda i: (0, 0)),
            pl.BlockSpec((128, 384), lambda i: (0, 0)),
            pl.BlockSpec((128, 384), lambda i: (0, 0)),
            pl.BlockSpec((1, 384), lambda i: (0, 0)),
            pl.BlockSpec((1, 384), lambda i: (0, 0)),
            pl.BlockSpec((128, 64), lambda i: (0, 0)),
            pl.BlockSpec((64, 64), lambda i: (0, 0)),
            pl.BlockSpec((1, 64), lambda i: (0, 0)),
            pl.BlockSpec((64, 128), lambda i: (0, 0)),
            pl.BlockSpec((128, 64), lambda i: (0, 0)),
            pl.BlockSpec((128, 64), lambda i: (0, 0)),
        ],
        out_specs=[
            pl.BlockSpec((BM, 128), lambda i: (i, 0)),
            pl.BlockSpec((BM, 64), lambda i: (i, 0)),
            pl.BlockSpec((BM, 64), lambda i: (i, 0)),
            pl.BlockSpec((BM, 1), lambda i: (i, 0)),
            pl.BlockSpec((1, 1), lambda i: (0, 0)),
        ],
        out_shape=[
            jax.ShapeDtypeStruct((VP, 128), f32),
            jax.ShapeDtypeStruct((VP, 64), f32),
            jax.ShapeDtypeStruct((VP, 64), f32),
            jax.ShapeDtypeStruct((VP, 1), f32),
            jax.ShapeDtypeStruct((1, 1), f32),
        ],
    )(s2, st, dvec, abias, bbias, obias, tpad, W3T, WihT, WhhT,
      bihr, bhhr, O1sT, O2T, ob2r, O3Tp, W1aT, W1bT)


def kernel(J_msg, b, msg_node, idx_msg_edge, target, W1, b1, W2, b2, W3, b3,
           Wih, Whh, bih, bhh, O1, ob1, O2, ob2, O3, ob3):
    del idx_msg_edge
    W1aT = W1[:, 0:128].T
    W1bT = W1[:, 132:260].T
    u = W1[:, 128] - W1[:, 129]
    v = W1[:, 261] - W1[:, 260]
    w = (W1[:, 130] - W1[:, 131]) + (W1[:, 263] - W1[:, 262])
    bp = jnp.pad(b, ((0, VP - V), (0, 0)))
    abias = bp * u[None, :]
    bbias = bp * v[None, :] + b1[None, :]
    obias = bp * (O1[:, 128] - O1[:, 129])[None, :] + ob1[None, :]
    tpad = jnp.pad(target, ((0, VP - V), (0, 0)), constant_values=1.0)
    W2T = W2.T
    W3T = W3.T
    WihT = Wih.T
    WhhT = Whh.T
    O1sT = O1[:, 0:128].T
    O2T = O2.T
    O3Tp = jnp.pad(O3.T, ((0, 0), (0, 128 - 2)))
    b2r = b2[None, :]
    bihr = bih[None, :]
    bhhr = bhh[None, :]
    ob2r = ob2[None, :]
    wr = w[None, :]
    ii = msg_node[:, 0].astype(jnp.int32)
    io = msg_node[:, 1].astype(jnp.int32)

    state = jnp.zeros((VP, H), f32)
    ap = abias
    bpp = bbias
    dvec = None
    ys = []
    lsum = None
    for t in range(NPROP):
        m1raw = _sc_gather(ap, bpp, ii, io)
        m2 = _tc_edge(m1raw, J_msg, W2T, b2r, wr)
        if t == 0:
            s2, deg2 = _sc_scatter_deg(m2, io)
            deg = deg2[0, :, 0] + deg2[1, :, 0]
            dvec = deg[:, None] * b3[None, :]
        else:
            s2 = _sc_scatter(m2, io)
        state, ap, bpp, y, lsum = _tc_node(
            s2, state, dvec, abias, bbias, obias, tpad, W3T, WihT, WhhT,
            bihr, bhhr, O1sT, O2T, ob2r, O3Tp, W1aT, W1bT)
        ys.append(y)
    y_step = jnp.concatenate(ys, axis=1)[:V, :]
    loss = (lsum[0, 0] / jnp.float32(V)).astype(f32)
    return (y_step, loss)


# speedup vs baseline: 5.2832x; 5.2832x over previous
import functools

import jax
import jax.numpy as jnp
from jax import lax
from jax.experimental import pallas as pl
from jax.experimental.pallas import tpu as pltpu
from jax.experimental.pallas import tpu_sc as plsc

V = 10000
E = 320000
H = 128
NPROP = 5
VP = 10240
NW = 32
EPW = E // NW
K = 400
NCH = EPW // K
RPT = VP // 16
BM = 1024
BE = 2000

f32 = jnp.float32


def _sc_mesh():
    return plsc.VectorSubcoreMesh(core_axis_name="c", subcore_axis_name="s")


_SC_PARAMS = pltpu.CompilerParams(use_tc_tiling_on_sc=False)


def _gather_body(ap, bp, ii, io, m1, ii_v, io_v, a_v, b_v, o_v, s1, s2):
    wid = lax.axis_index("s") * 2 + lax.axis_index("c")
    for ci in range(NCH):
        base = wid * EPW + ci * K
        pltpu.sync_copy(ii.at[pl.ds(base, K)], ii_v)
        pltpu.sync_copy(io.at[pl.ds(base, K)], io_v)
        ca = pltpu.async_copy(ap.at[ii_v], a_v, s1)
        cb = pltpu.async_copy(bp.at[io_v], b_v, s2)
        ca.wait()
        cb.wait()

        def row(r, carry):
            for cc in range(4):
                sl = pl.ds(16 * cc, 16)
                o_v[r, sl] = a_v[r, sl] + b_v[r, sl]
            return carry

        lax.fori_loop(0, K, row, 0)
        pltpu.sync_copy(o_v, m1.at[pl.ds(base, K)])


def _sc_gather(ap, bp, ii, io):
    gk = pl.kernel(
        _gather_body,
        out_type=jax.ShapeDtypeStruct((E, 64), f32),
        mesh=_sc_mesh(),
        compiler_params=_SC_PARAMS,
        scratch_types=[
            pltpu.VMEM((K,), jnp.int32),
            pltpu.VMEM((K,), jnp.int32),
            pltpu.VMEM((K, 64), f32),
            pltpu.VMEM((K, 64), f32),
            pltpu.VMEM((K, 64), f32),
            pltpu.SemaphoreType.DMA,
            pltpu.SemaphoreType.DMA,
        ],
    )
    return gk(ap, bp, ii, io)


def _zero_fill(buf, rows):
    def zrow(r, carry):
        for cc in range(buf.shape[1] // 16):
            buf[r, pl.ds(16 * cc, 16)] = jnp.zeros((16,), f32)
        return carry

    lax.fori_loop(0, rows, zrow, 0)


def _scatter_body_deg(m2, io, s2o, dego, m2_v, io_v, z_v, ones_v, zd_v, S_sh, D_sh):
    sid = lax.axis_index("s")
    cid = lax.axis_index("c")
    wid = sid * 2 + cid
    _zero_fill(z_v, 64)
    _zero_fill(zd_v, 64)

    def orow(r, carry):
        ones_v[r, pl.ds(0, 16)] = jnp.ones((16,), f32)
        return carry

    lax.fori_loop(0, K, orow, 0)
    for i in range(RPT // 64):
        pltpu.sync_copy(z_v, S_sh.at[pl.ds(sid * RPT + i * 64, 64)])
        pltpu.sync_copy(zd_v, D_sh.at[pl.ds(sid * RPT + i * 64, 64)])
    plsc.subcore_barrier()
    for ci in range(NCH):
        base = wid * EPW + ci * K
        pltpu.sync_copy(m2.at[pl.ds(base, K)], m2_v)
        pltpu.sync_copy(io.at[pl.ds(base, K)], io_v)
        pltpu.sync_copy(m2_v, S_sh.at[io_v], add=True)
        pltpu.sync_copy(ones_v, D_sh.at[io_v], add=True)
    plsc.subcore_barrier()
    pltpu.sync_copy(S_sh.at[pl.ds(sid * RPT, RPT)], s2o.at[cid, pl.ds(sid * RPT, RPT)])
    pltpu.sync_copy(D_sh.at[pl.ds(sid * RPT, RPT)], dego.at[cid, pl.ds(sid * RPT, RPT)])


def _scatter_body(m2, io, s2o, m2_v, io_v, z_v, S_sh):
    sid = lax.axis_index("s")
    cid = lax.axis_index("c")
    wid = sid * 2 + cid
    _zero_fill(z_v, 64)
    for i in range(RPT // 64):
        pltpu.sync_copy(z_v, S_sh.at[pl.ds(sid * RPT + i * 64, 64)])
    plsc.subcore_barrier()
    for ci in range(NCH):
        base = wid * EPW + ci * K
        pltpu.sync_copy(m2.at[pl.ds(base, K)], m2_v)
        pltpu.sync_copy(io.at[pl.ds(base, K)], io_v)
        pltpu.sync_copy(m2_v, S_sh.at[io_v], add=True)
    plsc.subcore_barrier()
    pltpu.sync_copy(S_sh.at[pl.ds(sid * RPT, RPT)], s2o.at[cid, pl.ds(sid * RPT, RPT)])


def _sc_scatter_deg(m2, io):
    sk = pl.kernel(
        _scatter_body_deg,
        out_type=(
            jax.ShapeDtypeStruct((2, VP, 64), f32),
            jax.ShapeDtypeStruct((2, VP, 16), f32),
        ),
        mesh=_sc_mesh(),
        compiler_params=_SC_PARAMS,
        scratch_types=[
            pltpu.VMEM((K, 64), f32),
            pltpu.VMEM((K,), jnp.int32),
            pltpu.VMEM((64, 64), f32),
            pltpu.VMEM((K, 16), f32),
            pltpu.VMEM((64, 16), f32),
            pltpu.VMEM_SHARED((VP, 64), f32),
            pltpu.VMEM_SHARED((VP, 16), f32),
        ],
    )
    return sk(m2, io)


def _sc_scatter(m2, io):
    sk = pl.kernel(
        _scatter_body,
        out_type=jax.ShapeDtypeStruct((2, VP, 64), f32),
        mesh=_sc_mesh(),
        compiler_params=_SC_PARAMS,
        scratch_types=[
            pltpu.VMEM((K, 64), f32),
            pltpu.VMEM((K,), jnp.int32),
            pltpu.VMEM((64, 64), f32),
            pltpu.VMEM_SHARED((VP, 64), f32),
        ],
    )
    return sk(m2, io)


def _edge_body(m1_ref, j_ref, w2_ref, b2_ref, w_ref, o_ref):
    x = m1_ref[...] + j_ref[...] * w_ref[...]
    x = jnp.maximum(x, 0.0)
    y = lax.dot_general(x, w2_ref[...], (((1,), (0,)), ((), ())),
                        preferred_element_type=f32) + b2_ref[...]
    o_ref[...] = jnp.maximum(y, 0.0)


def _tc_edge(m1raw, J, W2T, b2r, wr):
    return pl.pallas_call(
        _edge_body,
        grid=(E // BE,),
        in_specs=[
            pl.BlockSpec((BE, 64), lambda i: (i, 0)),
            pl.BlockSpec((BE, 1), lambda i: (i, 0)),
            pl.BlockSpec((64, 64), lambda i: (0, 0)),
            pl.BlockSpec((1, 64), lambda i: (0, 0)),
            pl.BlockSpec((1, 64), lambda i: (0, 0)),
        ],
        out_specs=pl.BlockSpec((BE, 64), lambda i: (i, 0)),
        out_shape=jax.ShapeDtypeStruct((E, 64), f32),
    )(m1raw, J, W2T, b2r, wr)


def _node_body(s2_ref, st_ref, dvec_ref, ab_ref, bb_ref, ob_ref, t_ref,
               w3t_ref, wiht_ref, whht_ref, bih_ref, bhh_ref,
               o1st_ref, o2t_ref, ob2_ref, o3tp_ref, w1at_ref, w1bt_ref,
               stn_ref, ap_ref, bp_ref, y_ref, l_ref):
    i = pl.program_id(0)
    s = s2_ref[0] + s2_ref[1]
    msg = lax.dot_general(s, w3t_ref[...], (((1,), (0,)), ((), ())),
                          preferred_element_type=f32) + dvec_ref[...]
    st = st_ref[...]
    gi = lax.dot_general(msg, wiht_ref[...], (((1,), (0,)), ((), ())),
                         preferred_element_type=f32) + bih_ref[...]
    gh = lax.dot_general(st, whht_ref[...], (((1,), (0,)), ((), ())),
                         preferred_element_type=f32) + bhh_ref[...]
    r = jax.nn.sigmoid(gi[:, 0:128] + gh[:, 0:128])
    z = jax.nn.sigmoid(gi[:, 128:256] + gh[:, 128:256])
    n = jnp.tanh(gi[:, 256:384] + r * gh[:, 256:384])
    stn = (1.0 - z) * n + z * st
    stn_ref[...] = stn
    o1 = lax.dot_general(stn, o1st_ref[...], (((1,), (0,)), ((), ())),
                         preferred_element_type=f32) + ob_ref[...]
    o1 = jnp.maximum(o1, 0.0)
    o2 = lax.dot_general(o1, o2t_ref[...], (((1,), (0,)), ((), ())),
                         preferred_element_type=f32) + ob2_ref[...]
    o2 = jnp.maximum(o2, 0.0)
    l01 = lax.dot_general(o2, o3tp_ref[...], (((1,), (0,)), ((), ())),
                          preferred_element_type=f32)
    l0 = l01[:, 0:1]
    l1 = l01[:, 1:2]
    m = jnp.maximum(l0, l1)
    lse = m + jnp.log(jnp.exp(l0 - m) + jnp.exp(l1 - m))
    y_ref[...] = jnp.exp(l0 - lse)
    ll = jnp.concatenate([l0 - lse, l1 - lse], axis=1)
    d = ll - jnp.log(t_ref[...])
    rows = i * BM + lax.broadcasted_iota(jnp.int32, (BM, 2), 0)
    sq = jnp.where(rows < V, d * d, 0.0)
    part = jnp.sum(sq, axis=(0, 1), keepdims=True)

    @pl.when(i == 0)
    def _():
        l_ref[...] = jnp.zeros((1, 1), f32)

    l_ref[...] += part
    ap_ref[...] = lax.dot_general(stn, w1at_ref[...], (((1,), (0,)), ((), ())),
                                  preferred_element_type=f32) + ab_ref[...]
    bp_ref[...] = lax.dot_general(stn, w1bt_ref[...], (((1,), (0,)), ((), ())),
                                  preferred_element_type=f32) + bb_ref[...]


def _tc_node(s2, st, dvec, abias, bbias, obias, tpad, W3T, WihT, WhhT,
             bihr, bhhr, O1sT, O2T, ob2r, O3Tp, W1aT, W1bT):
    return pl.pallas_call(
        _node_body,
        grid=(VP // BM,),
        in_specs=[
            pl.BlockSpec((2, BM, 64), lambda i: (0, i, 0)),
            pl.BlockSpec((BM, 128), lambda i: (i, 0)),
            pl.BlockSpec((BM, 128), lambda i: (i, 0)),
            pl.BlockSpec((BM, 64), lambda i: (i, 0)),
            pl.BlockSpec((BM, 64), lambda i: (i, 0)),
            pl.BlockSpec((BM, 64), lambda i: (i, 0)),
            pl.BlockSpec((BM, 2), lambda i: (i, 0)),
            pl.BlockSpec((64, 128), lambda i: (0, 0)),
            pl.BlockSpec((128, 384), lambda i: (0, 0)),
            pl.BlockSpec((128, 384), lambda i: (0, 0)),
            pl.BlockSpec((1, 384), lambda i: (0, 0)),
            pl.BlockSpec((1, 384), lambda i: (0, 0)),
            pl.BlockSpec((128, 64), lambda i: (0, 0)),
            pl.BlockSpec((64, 64), lambda i: (0, 0)),
            pl.BlockSpec((1, 64), lambda i: (0, 0)),
            pl.BlockSpec((64, 128), lambda i: (0, 0)),
            pl.BlockSpec((128, 64), lambda i: (0, 0)),
            pl.BlockSpec((128, 64), lambda i: (0, 0)),
        ],
        out_specs=[
            pl.BlockSpec((BM, 128), lambda i: (i, 0)),
            pl.BlockSpec((BM, 64), lambda i: (i, 0)),
            pl.BlockSpec((BM, 64), lambda i: (i, 0)),
            pl.BlockSpec((BM, 1), lambda i: (i, 0)),
            pl.BlockSpec((1, 1), lambda i: (0, 0)),
        ],
        out_shape=[
            jax.ShapeDtypeStruct((VP, 128), f32),
            jax.ShapeDtypeStruct((VP, 64), f32),
            jax.ShapeDtypeStruct((VP, 64), f32),
            jax.ShapeDtypeStruct((VP, 1), f32),
            jax.ShapeDtypeStruct((1, 1), f32),
        ],
    )(s2, st, dvec, abias, bbias, obias, tpad, W3T, WihT, WhhT,
      bihr, bhhr, O1sT, O2T, ob2r, O3Tp, W1aT, W1bT)


def kernel(J_msg, b, msg_node, idx_msg_edge, target, W1, b1, W2, b2, W3, b3,
           Wih, Whh, bih, bhh, O1, ob1, O2, ob2, O3, ob3):
    del idx_msg_edge
    W1aT = W1[:, 0:128].T
    W1bT = W1[:, 132:260].T
    u = W1[:, 128] - W1[:, 129]
    v = W1[:, 261] - W1[:, 260]
    w = (W1[:, 130] - W1[:, 131]) + (W1[:, 263] - W1[:, 262])
    bp = jnp.pad(b, ((0, VP - V), (0, 0)))
    abias = bp * u[None, :]
    bbias = bp * v[None, :] + b1[None, :]
    obias = bp * (O1[:, 128] - O1[:, 129])[None, :] + ob1[None, :]
    tpad = jnp.pad(target, ((0, VP - V), (0, 0)), constant_values=1.0)
    W2T = W2.T
    W3T = W3.T
    WihT = Wih.T
    WhhT = Whh.T
    O1sT = O1[:, 0:128].T
    O2T = O2.T
    O3Tp = jnp.pad(O3.T, ((0, 0), (0, 128 - 2)))
    b2r = b2[None, :]
    bihr = bih[None, :]
    bhhr = bhh[None, :]
    ob2r = ob2[None, :]
    wr = w[None, :]
    ii = msg_node[:, 0].astype(jnp.int32)
    io = msg_node[:, 1].astype(jnp.int32)

    state = jnp.zeros((VP, H), f32)
    ap = abias
    bpp = bbias
    dvec = None
    ys = []
    lsum = None
    for t in range(NPROP):
        m1raw = _sc_gather(ap, bpp, ii, io)
        m2 = _tc_edge(m1raw, J_msg, W2T, b2r, wr)
        if t == 0:
            s2, deg2 = _sc_scatter_deg(m2, io)
            deg = deg2[0, :, 0] + deg2[1, :, 0]
            dvec = deg[:, None] * b3[None, :]
        else:
            s2 = _sc_scatter(m2, io)
        state, ap, bpp, y, lsum = _tc_node(
            s2, state, dvec, abias, bbias, obias, tpad, W3T, WihT, WhhT,
            bihr, bhhr, O1sT, O2T, ob2r, O3Tp, W1aT, W1bT)
        ys.append(y)
    y_step = jnp.concatenate(ys, axis=1)[:V, :]
    loss = (lsum[0, 0] / jnp.float32(V)).astype(f32)
    return (y_step, loss)
